# Initial kernel scaffold; baseline (speedup 1.0000x reference)
import jax
import jax.numpy as jnp
from jax.experimental import pallas as pl


def kernel(x, gate_W, gate_b, W1, b1, W2, b2):
    raise NotImplementedError("write your pallas kernel here")


# speedup vs baseline: 2.3426x; 2.3426x over previous
import functools

import jax
import jax.numpy as jnp
from jax import lax
from jax.experimental import pallas as pl
from jax.experimental.pallas import tpu as pltpu

DIM = 1024
E = 8
HID = 2048
N = 2048
EP = 128
T = 256
G = 16
ROWS = G * T


def _gate_body(x_ref, w_ref, b_ref, idx_ref, val_ref):
    s = jnp.dot(x_ref[...], w_ref[...], preferred_element_type=jnp.float32)
    s = s + b_ref[...]
    m = jnp.max(s, axis=-1, keepdims=True)
    lane = lax.broadcasted_iota(jnp.int32, s.shape, 1)
    cand = jnp.where(s >= m, lane, EP)
    idx_ref[...] = jnp.min(cand, axis=-1, keepdims=True)
    val_ref[...] = m


def _gate(x2, gate_W, gate_b):
    wp = jnp.zeros((DIM, EP), jnp.float32).at[:, :E].set(gate_W)
    bp = jnp.full((1, EP), -1e30, jnp.float32).at[0, :E].set(gate_b)
    bt = 256
    idx, val = pl.pallas_call(
        _gate_body,
        grid=(N // bt,),
        in_specs=[
            pl.BlockSpec((bt, DIM), lambda g: (g, 0)),
            pl.BlockSpec((DIM, EP), lambda g: (0, 0)),
            pl.BlockSpec((1, EP), lambda g: (0, 0)),
        ],
        out_specs=[
            pl.BlockSpec((bt, 1), lambda g: (g, 0)),
            pl.BlockSpec((bt, 1), lambda g: (g, 0)),
        ],
        out_shape=[
            jax.ShapeDtypeStruct((N, 1), jnp.int32),
            jax.ShapeDtypeStruct((N, 1), jnp.float32),
        ],
    )(x2, wp, bp)
    return idx.reshape(N), val.reshape(N)


def _ffn_body(te_ref, xp_ref, w1_ref, b1_ref, w2_ref, b2_ref, sc_ref, out_ref):
    h = jnp.dot(xp_ref[...], w1_ref[0], preferred_element_type=jnp.float32)
    h = h + b1_ref[0]
    h = 0.5 * h * (1.0 + lax.erf(h * 0.7071067811865476))
    y = jnp.dot(h, w2_ref[0], preferred_element_type=jnp.float32)
    out_ref[...] = (y + b2_ref[0]) * sc_ref[...]


def _ffn(tile_expert, x_pad, W1, b1, W2, b2, score_pad):
    return pl.pallas_call(
        _ffn_body,
        grid_spec=pltpu.PrefetchScalarGridSpec(
            num_scalar_prefetch=1,
            grid=(G,),
            in_specs=[
                pl.BlockSpec((T, DIM), lambda g, te: (g, 0)),
                pl.BlockSpec((1, DIM, HID), lambda g, te: (te[g], 0, 0)),
                pl.BlockSpec((1, 1, HID), lambda g, te: (te[g], 0, 0)),
                pl.BlockSpec((1, HID, DIM), lambda g, te: (te[g], 0, 0)),
                pl.BlockSpec((1, 1, DIM), lambda g, te: (te[g], 0, 0)),
                pl.BlockSpec((T, 1), lambda g, te: (g, 0)),
            ],
            out_specs=pl.BlockSpec((T, DIM), lambda g, te: (g, 0)),
        ),
        out_shape=jax.ShapeDtypeStruct((ROWS, DIM), jnp.float32),
        compiler_params=pltpu.CompilerParams(
            dimension_semantics=("arbitrary",),
            vmem_limit_bytes=100 * 1024 * 1024,
        ),
    )(tile_expert, x_pad, W1, b1.reshape(E, 1, HID), W2,
      b2.reshape(E, 1, DIM), score_pad)


def kernel(x, gate_W, gate_b, W1, b1, W2, b2):
    x2 = x.reshape(N, DIM)
    idx, score = _gate(x2, gate_W, gate_b)

    oh = (idx[:, None] == jnp.arange(E, dtype=jnp.int32)[None, :]).astype(jnp.int32)
    counts = oh.sum(axis=0)
    rank = jnp.take_along_axis(jnp.cumsum(oh, axis=0), idx[:, None], axis=1)[:, 0] - 1
    ntiles = (counts + T - 1) // T
    tile_off = jnp.concatenate([jnp.zeros((1,), jnp.int32),
                                jnp.cumsum(ntiles).astype(jnp.int32)])
    dst = tile_off[idx] * T + rank
    src_pad = jnp.zeros((ROWS,), jnp.int32).at[dst].set(
        jnp.arange(N, dtype=jnp.int32))
    g_ids = jnp.arange(G, dtype=jnp.int32)
    te = jnp.minimum(
        jnp.searchsorted(tile_off[1:], g_ids, side="right").astype(jnp.int32),
        E - 1)

    x_pad = x2[src_pad]
    score_pad = score[src_pad].reshape(ROWS, 1)

    y_pad = _ffn(te, x_pad, W1, b1, W2, b2, score_pad)

    out = y_pad[dst]
    return out.reshape(1, N, DIM)


# speedup vs baseline: 2.3447x; 1.0009x over previous
import functools

import jax
import jax.numpy as jnp
from jax import lax
from jax.experimental import pallas as pl
from jax.experimental.pallas import tpu as pltpu

DIM = 1024
E = 8
HID = 2048
N = 2048
EP = 128
T = 256
G = 16
ROWS = G * T


def _gate_body(x_ref, w_ref, b_ref, idx_ref, val_ref):
    s = jnp.dot(x_ref[...], w_ref[...], preferred_element_type=jnp.float32)
    s = s + b_ref[...]
    m = jnp.max(s, axis=-1, keepdims=True)
    lane = lax.broadcasted_iota(jnp.int32, s.shape, 1)
    cand = jnp.where(s >= m, lane, EP)
    idx_ref[...] = jnp.min(cand, axis=-1, keepdims=True)
    val_ref[...] = m


def _gate(x2, gate_W, gate_b):
    wp = jnp.zeros((DIM, EP), jnp.float32).at[:, :E].set(gate_W)
    bp = jnp.full((1, EP), -1e30, jnp.float32).at[0, :E].set(gate_b)
    bt = 256
    idx, val = pl.pallas_call(
        _gate_body,
        grid=(N // bt,),
        in_specs=[
            pl.BlockSpec((bt, DIM), lambda g: (g, 0)),
            pl.BlockSpec((DIM, EP), lambda g: (0, 0)),
            pl.BlockSpec((1, EP), lambda g: (0, 0)),
        ],
        out_specs=[
            pl.BlockSpec((bt, 1), lambda g: (g, 0)),
            pl.BlockSpec((bt, 1), lambda g: (g, 0)),
        ],
        out_shape=[
            jax.ShapeDtypeStruct((N, 1), jnp.int32),
            jax.ShapeDtypeStruct((N, 1), jnp.float32),
        ],
    )(x2, wp, bp)
    return idx.reshape(N), val.reshape(N)


def _ffn_body(te_ref, xp_ref, w1_ref, b1_ref, w2_ref, b2_ref, sc_ref, out_ref):
    xb = xp_ref[...].astype(jnp.bfloat16)
    h = jnp.dot(xb, w1_ref[0].astype(jnp.bfloat16),
                preferred_element_type=jnp.float32)
    h = h + b1_ref[0]
    h = 0.5 * h * (1.0 + lax.erf(h * 0.7071067811865476))
    y = jnp.dot(h.astype(jnp.bfloat16), w2_ref[0].astype(jnp.bfloat16),
                preferred_element_type=jnp.float32)
    out_ref[...] = (y + b2_ref[0]) * sc_ref[...]


def _ffn(tile_expert, x_pad, W1, b1, W2, b2, score_pad):
    return pl.pallas_call(
        _ffn_body,
        grid_spec=pltpu.PrefetchScalarGridSpec(
            num_scalar_prefetch=1,
            grid=(G,),
            in_specs=[
                pl.BlockSpec((T, DIM), lambda g, te: (g, 0)),
                pl.BlockSpec((1, DIM, HID), lambda g, te: (te[g], 0, 0)),
                pl.BlockSpec((1, 1, HID), lambda g, te: (te[g], 0, 0)),
                pl.BlockSpec((1, HID, DIM), lambda g, te: (te[g], 0, 0)),
                pl.BlockSpec((1, 1, DIM), lambda g, te: (te[g], 0, 0)),
                pl.BlockSpec((T, 1), lambda g, te: (g, 0)),
            ],
            out_specs=pl.BlockSpec((T, DIM), lambda g, te: (g, 0)),
        ),
        out_shape=jax.ShapeDtypeStruct((ROWS, DIM), jnp.float32),
        compiler_params=pltpu.CompilerParams(
            dimension_semantics=("arbitrary",),
            vmem_limit_bytes=100 * 1024 * 1024,
        ),
    )(tile_expert, x_pad, W1, b1.reshape(E, 1, HID), W2,
      b2.reshape(E, 1, DIM), score_pad)


def kernel(x, gate_W, gate_b, W1, b1, W2, b2):
    x2 = x.reshape(N, DIM)
    idx, score = _gate(x2, gate_W, gate_b)

    oh = (idx[:, None] == jnp.arange(E, dtype=jnp.int32)[None, :]).astype(jnp.int32)
    counts = oh.sum(axis=0)
    rank = jnp.take_along_axis(jnp.cumsum(oh, axis=0), idx[:, None], axis=1)[:, 0] - 1
    ntiles = (counts + T - 1) // T
    tile_off = jnp.concatenate([jnp.zeros((1,), jnp.int32),
                                jnp.cumsum(ntiles).astype(jnp.int32)])
    dst = tile_off[idx] * T + rank
    src_pad = jnp.zeros((ROWS,), jnp.int32).at[dst].set(
        jnp.arange(N, dtype=jnp.int32))
    g_ids = jnp.arange(G, dtype=jnp.int32)
    te = jnp.minimum(
        jnp.searchsorted(tile_off[1:], g_ids, side="right").astype(jnp.int32),
        E - 1)

    x_pad = x2[src_pad]
    score_pad = score[src_pad].reshape(ROWS, 1)

    y_pad = _ffn(te, x_pad, W1, b1, W2, b2, score_pad)

    out = y_pad[dst]
    return out.reshape(1, N, DIM)
